# baseline (speedup 1.0000x reference)
import functools

import jax
import jax.numpy as jnp
from jax import lax
from jax.experimental import pallas as pl
from jax.experimental.pallas import tpu as pltpu
from jax.experimental.pallas import tpu_sc as plsc

T = 2048
D = 1024
E = 8
TK = 2
F = 512
FS = 1024
EPS = 1e-6

BT = 256
BM = 256
NP = T * TK + E * BM
NB = NP // BM

NC, NS = 2, 16
NW = NC * NS

TPW = T // NW
PPW = T * TK // NW
GC = 64
GNCH = PPW // GC
SC_ = 32
SNCH = TPW // SC_


def _k1a_body(hs_ref, nw_ref, gw_ref, i1_ref, i2_ref, p1_ref, p2_ref):
    x = hs_ref[...]
    var = jnp.mean(x * x, axis=-1, keepdims=True)
    nx = nw_ref[...] * (x * lax.rsqrt(var + EPS))

    logits = lax.dot_general(nx, gw_ref[...], (((1,), (1,)), ((), ())))
    eio = lax.broadcasted_iota(jnp.int32, logits.shape, 1)
    m1 = jnp.max(logits, axis=-1)
    i1 = jnp.min(jnp.where(logits == m1[:, None], eio, E), axis=-1)
    masked = jnp.where(eio == i1[:, None], -jnp.inf, logits)
    m2 = jnp.max(masked, axis=-1)
    i2 = jnp.min(jnp.where(masked == m2[:, None], eio, E), axis=-1)
    t = jnp.exp(m2 - m1)
    i1_ref[0, 0, :] = i1
    i2_ref[0, 0, :] = i2
    p1_ref[0, 0, :] = 1.0 / (1.0 + t)
    p2_ref[0, 0, :] = t / (1.0 + t)


_NBT = T // BT
_k1a = pl.pallas_call(
    _k1a_body,
    grid=(_NBT,),
    in_specs=[
        pl.BlockSpec((BT, D), lambda b: (b, 0)),
        pl.BlockSpec((1, D), lambda b: (0, 0)),
        pl.BlockSpec((E, D), lambda b: (0, 0)),
    ],
    out_specs=[
        pl.BlockSpec((1, 1, BT), lambda b: (b, 0, 0)),
        pl.BlockSpec((1, 1, BT), lambda b: (b, 0, 0)),
        pl.BlockSpec((1, 1, BT), lambda b: (b, 0, 0)),
        pl.BlockSpec((1, 1, BT), lambda b: (b, 0, 0)),
    ],
    out_shape=[
        jax.ShapeDtypeStruct((_NBT, 1, BT), jnp.int32),
        jax.ShapeDtypeStruct((_NBT, 1, BT), jnp.int32),
        jax.ShapeDtypeStruct((_NBT, 1, BT), jnp.float32),
        jax.ShapeDtypeStruct((_NBT, 1, BT), jnp.float32),
    ],
)


def _k1b_body(hs_ref, sw1_ref, sw2_ref, swo_ref, base_ref):
    x = hs_ref[...]
    dn = (((1,), (1,)), ((), ()))
    s1 = lax.dot_general(x, sw1_ref[...], dn)
    s2 = lax.dot_general(x, sw2_ref[...], dn)
    inter = s1 * (s2 * jax.nn.sigmoid(s2))
    sh = lax.dot_general(inter, swo_ref[...], dn)
    base_ref[...] = x + sh


_k1b = pl.pallas_call(
    _k1b_body,
    grid=(_NBT,),
    in_specs=[
        pl.BlockSpec((BT, D), lambda b: (b, 0)),
        pl.BlockSpec((FS, D), lambda b: (0, 0)),
        pl.BlockSpec((FS, D), lambda b: (0, 0)),
        pl.BlockSpec((D, FS), lambda b: (0, 0)),
    ],
    out_specs=pl.BlockSpec((BT, D), lambda b: (b, 0)),
    out_shape=jax.ShapeDtypeStruct((T, D), jnp.float32),
)


def _k15_body(e1_ref, e2_ref, d0_ref, d1_ref, dga_ref, be_ref):
    f32 = jnp.float32
    hi = lax.Precision.HIGHEST
    E1 = e1_ref[...].astype(f32)
    E2 = e2_ref[...].astype(f32)

    spread = (lax.broadcasted_iota(jnp.int32, (32, 512), 0)
              == lax.broadcasted_iota(jnp.int32, (32, 512), 1) // 16
              ).astype(f32)
    e1rep = jnp.dot(E1, spread, precision=hi)
    e2rep = jnp.dot(E2, spread, precision=hi)
    col = lax.broadcasted_iota(jnp.int32, (64, 512), 1)
    esel = jnp.where((col // 8) % 2 == 1, e2rep, e1rep)
    oh = (esel == (col % 8).astype(f32)).astype(f32)

    tri = (lax.broadcasted_iota(jnp.int32, (64, 64), 0)
           >= lax.broadcasted_iota(jnp.int32, (64, 64), 1)).astype(f32)
    rank_inc = jnp.dot(tri, oh, precision=hi)
    tot = rank_inc[63:64, :]

    cc0 = lax.broadcasted_iota(jnp.int32, (512, 512), 0)
    cc1 = lax.broadcasted_iota(jnp.int32, (512, 512), 1)
    same_e = (cc0 % 8) == (cc1 % 8)
    carry = jnp.dot(tot, (same_e & ((cc0 // 8) < (cc1 // 8))).astype(f32),
                    precision=hi)
    counts = jnp.dot(tot, same_e.astype(f32), precision=hi)
    counts_i = (counts + 0.5).astype(jnp.int32)
    pcf = (((counts_i + BM - 1) // BM) * BM).astype(f32)
    cpc = jnp.dot(pcf, ((cc0 < 8) & ((cc0 % 8) <= (cc1 % 8))).astype(f32),
                  precision=hi)
    po = cpc - pcf

    destsum = oh * (po + carry + rank_inc - 1.0)
    grp = ((lax.broadcasted_iota(jnp.int32, (512, 64), 0) // 8)
           == lax.broadcasted_iota(jnp.int32, (512, 64), 1)).astype(f32)
    dest = jnp.dot(destsum, grp, precision=hi)

    s0 = lax.broadcasted_iota(jnp.int32, (64, 32), 0)
    s1 = lax.broadcasted_iota(jnp.int32, (64, 32), 1)
    d_e = jnp.dot(dest, (s0 == 2 * s1).astype(f32), precision=hi)
    d_o = jnp.dot(dest, (s0 == 2 * s1 + 1).astype(f32), precision=hi)
    d0_ref[...] = (d_e + 0.5).astype(jnp.int32).reshape(NW, SNCH, SC_)
    d1_ref[...] = (d_o + 0.5).astype(jnp.int32).reshape(NW, SNCH, SC_)
    dga_ref[...] = (dest + 0.5).astype(jnp.int32).reshape(NW, GNCH, GC)

    bsub = (lax.broadcasted_iota(jnp.int32, (32, 512), 0) * BM).astype(f32)
    lane = lax.broadcasted_iota(jnp.int32, (32, 512), 1)
    ge = ((jnp.broadcast_to(cpc, (32, 512)) <= bsub)
          & (lane < 8)).astype(f32)
    be = jnp.sum(ge, axis=1, keepdims=True)
    bei = jnp.minimum((be + 0.5).astype(jnp.int32), E - 1)
    fill = jnp.broadcast_to(po + counts, (32, 512))
    ohb = ((lane % 8) == jnp.broadcast_to(bei, (32, 512))) & (lane < 8)
    fillsel = jnp.sum(jnp.where(ohb, fill, 0.0), axis=1, keepdims=True)
    skip = (bsub[:, :1] >= fillsel).astype(jnp.int32)
    be_ref[...] = bei + E * skip


_k15 = pl.pallas_call(
    _k15_body,
    out_shape=[
        jax.ShapeDtypeStruct((NW, SNCH, SC_), jnp.int32),
        jax.ShapeDtypeStruct((NW, SNCH, SC_), jnp.int32),
        jax.ShapeDtypeStruct((NW, GNCH, GC), jnp.int32),
        jax.ShapeDtypeStruct((32, 1), jnp.int32),
    ],
)


def _dispatch_body(normed_hbm, d0_hbm, d1_hbm, out_hbm,
                   idx0_v, idx1_v, rows_v, sem):
    wid = lax.axis_index("s") * NC + lax.axis_index("c")
    pltpu.sync_copy(d0_hbm.at[wid], idx0_v)
    pltpu.sync_copy(d1_hbm.at[wid], idx1_v)
    pltpu.sync_copy(normed_hbm.at[pl.ds(wid * TPW, TPW)], rows_v)
    cps = []
    for c in range(SNCH):
        src = rows_v.at[pl.ds(c * SC_, SC_)]
        cps.append(pltpu.async_copy(src, out_hbm.at[idx0_v.at[c]], sem))
        cps.append(pltpu.async_copy(src, out_hbm.at[idx1_v.at[c]], sem))
    for cp in cps:
        cp.wait()


@functools.lru_cache(maxsize=None)
def _make_k2():
    return functools.partial(
        pl.kernel,
        out_type=jax.ShapeDtypeStruct((NP, D), jnp.float32),
        mesh=plsc.VectorSubcoreMesh(core_axis_name="c", subcore_axis_name="s",
                                    num_cores=NC, num_subcores=NS),
        scratch_types=[
            pltpu.VMEM((SNCH, SC_), jnp.int32),
            pltpu.VMEM((SNCH, SC_), jnp.int32),
            pltpu.VMEM((TPW, D), jnp.float32),
            pltpu.SemaphoreType.DMA,
        ],
    )(_dispatch_body)


def _k3_body(be_ref, x_ref, nw_ref, w1_ref, w2_ref, wo_ref, o_ref):
    raw = be_ref[pl.program_id(0)]
    e = lax.rem(raw, E)

    @pl.when(raw < E)
    def _():
        xr = x_ref[...]
        var = jnp.mean(xr * xr, axis=-1, keepdims=True)
        x = nw_ref[...] * (xr * lax.rsqrt(var + EPS))
        dn = (((1,), (1,)), ((), ()))
        h1 = lax.dot_general(x, w1_ref[e], dn)
        h2 = lax.dot_general(x, w2_ref[e], dn)
        inter = h1 * (h2 * jax.nn.sigmoid(h2))
        o_ref[...] = lax.dot_general(inter, wo_ref[e], dn)


_k3 = pl.pallas_call(
    _k3_body,
    grid_spec=pltpu.PrefetchScalarGridSpec(
        num_scalar_prefetch=1,
        grid=(NB,),
        in_specs=[
            pl.BlockSpec((BM, D), lambda b, be: (jnp.where(be[b] >= E, 0, b), 0)),
            pl.BlockSpec((1, D), lambda b, be: (0, 0)),
            pl.BlockSpec((E, F, D), lambda b, be: (0, 0, 0)),
            pl.BlockSpec((E, F, D), lambda b, be: (0, 0, 0)),
            pl.BlockSpec((E, D, F), lambda b, be: (0, 0, 0)),
        ],
        out_specs=pl.BlockSpec((BM, D), lambda b, be: (b, 0)),
    ),
    out_shape=jax.ShapeDtypeStruct((NP, D), jnp.float32),
)


def _pairgather_body(rows_hbm, dga_hbm, out_hbm, idx_v, rows_v, sem):
    wid = lax.axis_index("s") * NC + lax.axis_index("c")
    pltpu.sync_copy(dga_hbm.at[wid], idx_v)
    base = wid * PPW
    for c in range(GNCH):
        pltpu.async_copy(rows_hbm.at[idx_v.at[c]], rows_v, sem).wait()
        pltpu.sync_copy(rows_v, out_hbm.at[pl.ds(base + c * GC, GC)])


@functools.lru_cache(maxsize=None)
def _make_k4():
    return functools.partial(
        pl.kernel,
        out_type=jax.ShapeDtypeStruct((T * TK, D), jnp.float32),
        mesh=plsc.VectorSubcoreMesh(core_axis_name="c", subcore_axis_name="s",
                                    num_cores=NC, num_subcores=NS),
        scratch_types=[
            pltpu.VMEM((GNCH, GC), jnp.int32),
            pltpu.VMEM((GC, D), jnp.float32),
            pltpu.SemaphoreType.DMA,
        ],
    )(_pairgather_body)


def _k5_body(pairs_ref, base_ref, p1_ref, p2_ref, o_ref):
    ident = (lax.broadcasted_iota(jnp.int32, (BT, BT), 0)
             == lax.broadcasted_iota(jnp.int32, (BT, BT), 1)
             ).astype(jnp.float32)
    dnl = (((1,), (1,)), ((), ()))
    w0 = lax.dot_general(ident, p1_ref[...].reshape(1, BT), dnl,
                         precision=lax.Precision.HIGHEST)
    w1 = lax.dot_general(ident, p2_ref[...].reshape(1, BT), dnl,
                         precision=lax.Precision.HIGHEST)
    xr = pairs_ref[...].reshape(BT, TK, D)
    o_ref[...] = base_ref[...] + w0 * xr[:, 0, :] + w1 * xr[:, 1, :]


_k5 = pl.pallas_call(
    _k5_body,
    grid=(_NBT,),
    in_specs=[
        pl.BlockSpec((TK * BT, D), lambda b: (b, 0)),
        pl.BlockSpec((BT, D), lambda b: (b, 0)),
        pl.BlockSpec((1, 1, BT), lambda b: (b, 0, 0)),
        pl.BlockSpec((1, 1, BT), lambda b: (b, 0, 0)),
    ],
    out_specs=pl.BlockSpec((BT, D), lambda b: (b, 0)),
    out_shape=jax.ShapeDtypeStruct((T, D), jnp.float32),
)


@jax.jit
def kernel(hidden_states, norm_w, gate_w, w1, w2, wo, sw1, sw2, swo):
    flat = hidden_states.reshape(T, D)
    nw2 = norm_w.reshape(1, D)
    i1o, i2o, p1o, p2o = _k1a(flat, nw2, gate_w)
    base = _k1b(flat, sw1, sw2, swo)

    d0, d1, dga, beo = _k15(i1o.reshape(64, 32), i2o.reshape(64, 32))
    block_e = beo[:NB, 0]

    dispatched = _make_k2()(flat, d0, d1)
    out_rows = _k3(block_e, dispatched, nw2, w1, w2, wo)
    pairs = _make_k4()(out_rows, dga)
    y = _k5(pairs, base, p1o, p2o)
    return y.reshape(hidden_states.shape)


# speedup vs baseline: 1.0126x; 1.0126x over previous
import functools

import jax
import jax.numpy as jnp
from jax import lax
from jax.experimental import pallas as pl
from jax.experimental.pallas import tpu as pltpu
from jax.experimental.pallas import tpu_sc as plsc

T = 2048
D = 1024
E = 8
TK = 2
F = 512
FS = 1024
EPS = 1e-6

BT = 256
BM = 256
NP = T * TK + E * BM
NB = NP // BM

NC, NS = 2, 16
NW = NC * NS

TPW = T // NW
PPW = T * TK // NW
GC = 64
GNCH = PPW // GC
SC_ = 32
SNCH = TPW // SC_


def _k1a_body(hs_ref, nw_ref, gw_ref, i1_ref, i2_ref, p1_ref, p2_ref):
    x = hs_ref[...]
    var = jnp.mean(x * x, axis=-1, keepdims=True)
    nx = nw_ref[...] * (x * lax.rsqrt(var + EPS))

    logits = lax.dot_general(nx, gw_ref[...], (((1,), (1,)), ((), ())))
    eio = lax.broadcasted_iota(jnp.int32, logits.shape, 1)
    m1 = jnp.max(logits, axis=-1)
    i1 = jnp.min(jnp.where(logits == m1[:, None], eio, E), axis=-1)
    masked = jnp.where(eio == i1[:, None], -jnp.inf, logits)
    m2 = jnp.max(masked, axis=-1)
    i2 = jnp.min(jnp.where(masked == m2[:, None], eio, E), axis=-1)
    t = jnp.exp(m2 - m1)
    i1_ref[0, 0, :] = i1
    i2_ref[0, 0, :] = i2
    p1_ref[0, 0, :] = 1.0 / (1.0 + t)
    p2_ref[0, 0, :] = t / (1.0 + t)


_NBT = T // BT
_k1a = pl.pallas_call(
    _k1a_body,
    grid=(_NBT,),
    in_specs=[
        pl.BlockSpec((BT, D), lambda b: (b, 0)),
        pl.BlockSpec((1, D), lambda b: (0, 0)),
        pl.BlockSpec((E, D), lambda b: (0, 0)),
    ],
    out_specs=[
        pl.BlockSpec((1, 1, BT), lambda b: (b, 0, 0)),
        pl.BlockSpec((1, 1, BT), lambda b: (b, 0, 0)),
        pl.BlockSpec((1, 1, BT), lambda b: (b, 0, 0)),
        pl.BlockSpec((1, 1, BT), lambda b: (b, 0, 0)),
    ],
    out_shape=[
        jax.ShapeDtypeStruct((_NBT, 1, BT), jnp.int32),
        jax.ShapeDtypeStruct((_NBT, 1, BT), jnp.int32),
        jax.ShapeDtypeStruct((_NBT, 1, BT), jnp.float32),
        jax.ShapeDtypeStruct((_NBT, 1, BT), jnp.float32),
    ],
)


def _k1b_body(hs_ref, sw1_ref, sw2_ref, swo_ref, base_ref):
    x = hs_ref[...]
    dn = (((1,), (1,)), ((), ()))
    s1 = lax.dot_general(x, sw1_ref[...], dn)
    s2 = lax.dot_general(x, sw2_ref[...], dn)
    inter = s1 * (s2 * jax.nn.sigmoid(s2))
    sh = lax.dot_general(inter, swo_ref[...], dn)
    base_ref[...] = x + sh


_k1b = pl.pallas_call(
    _k1b_body,
    grid=(_NBT,),
    in_specs=[
        pl.BlockSpec((BT, D), lambda b: (b, 0)),
        pl.BlockSpec((FS, D), lambda b: (0, 0)),
        pl.BlockSpec((FS, D), lambda b: (0, 0)),
        pl.BlockSpec((D, FS), lambda b: (0, 0)),
    ],
    out_specs=pl.BlockSpec((BT, D), lambda b: (b, 0)),
    out_shape=jax.ShapeDtypeStruct((T, D), jnp.float32),
)


def _k15_body(e1_ref, e2_ref, d0_ref, d1_ref, dga_ref, be_ref):
    f32 = jnp.float32
    hi = lax.Precision.HIGHEST
    E1 = e1_ref[...].astype(f32)
    E2 = e2_ref[...].astype(f32)

    spread = (lax.broadcasted_iota(jnp.int32, (32, 512), 0)
              == lax.broadcasted_iota(jnp.int32, (32, 512), 1) // 16
              ).astype(f32)
    e1rep = jnp.dot(E1, spread, precision=hi)
    e2rep = jnp.dot(E2, spread, precision=hi)
    col = lax.broadcasted_iota(jnp.int32, (64, 512), 1)
    esel = jnp.where((col // 8) % 2 == 1, e2rep, e1rep)
    oh = (esel == (col % 8).astype(f32)).astype(f32)

    tri = (lax.broadcasted_iota(jnp.int32, (64, 64), 0)
           >= lax.broadcasted_iota(jnp.int32, (64, 64), 1)).astype(f32)
    rank_inc = jnp.dot(tri, oh, precision=hi)
    tot = rank_inc[63:64, :]

    cc0 = lax.broadcasted_iota(jnp.int32, (512, 512), 0)
    cc1 = lax.broadcasted_iota(jnp.int32, (512, 512), 1)
    same_e = (cc0 % 8) == (cc1 % 8)
    carry = jnp.dot(tot, (same_e & ((cc0 // 8) < (cc1 // 8))).astype(f32),
                    precision=hi)
    counts = jnp.dot(tot, same_e.astype(f32), precision=hi)
    counts_i = (counts + 0.5).astype(jnp.int32)
    pcf = (((counts_i + BM - 1) // BM) * BM).astype(f32)
    cpc = jnp.dot(pcf, ((cc0 < 8) & ((cc0 % 8) <= (cc1 % 8))).astype(f32),
                  precision=hi)
    po = cpc - pcf

    destsum = oh * (po + carry + rank_inc - 1.0)
    grp = ((lax.broadcasted_iota(jnp.int32, (512, 64), 0) // 8)
           == lax.broadcasted_iota(jnp.int32, (512, 64), 1)).astype(f32)
    dest = jnp.dot(destsum, grp, precision=hi)

    s0 = lax.broadcasted_iota(jnp.int32, (64, 32), 0)
    s1 = lax.broadcasted_iota(jnp.int32, (64, 32), 1)
    d_e = jnp.dot(dest, (s0 == 2 * s1).astype(f32), precision=hi)
    d_o = jnp.dot(dest, (s0 == 2 * s1 + 1).astype(f32), precision=hi)
    d0_ref[...] = (d_e + 0.5).astype(jnp.int32).reshape(NW, SNCH, SC_)
    d1_ref[...] = (d_o + 0.5).astype(jnp.int32).reshape(NW, SNCH, SC_)
    dga_ref[...] = (dest + 0.5).astype(jnp.int32).reshape(NW, GNCH, GC)

    bsub = (lax.broadcasted_iota(jnp.int32, (32, 512), 0) * BM).astype(f32)
    lane = lax.broadcasted_iota(jnp.int32, (32, 512), 1)
    ge = ((jnp.broadcast_to(cpc, (32, 512)) <= bsub)
          & (lane < 8)).astype(f32)
    be = jnp.sum(ge, axis=1, keepdims=True)
    bei = jnp.minimum((be + 0.5).astype(jnp.int32), E - 1)
    fill = jnp.broadcast_to(po + counts, (32, 512))
    ohb = ((lane % 8) == jnp.broadcast_to(bei, (32, 512))) & (lane < 8)
    fillsel = jnp.sum(jnp.where(ohb, fill, 0.0), axis=1, keepdims=True)
    skip = (bsub[:, :1] >= fillsel).astype(jnp.int32)
    be_ref[...] = bei + E * skip


_k15 = pl.pallas_call(
    _k15_body,
    out_shape=[
        jax.ShapeDtypeStruct((NW, SNCH, SC_), jnp.int32),
        jax.ShapeDtypeStruct((NW, SNCH, SC_), jnp.int32),
        jax.ShapeDtypeStruct((NW, GNCH, GC), jnp.int32),
        jax.ShapeDtypeStruct((32, 1), jnp.int32),
    ],
)


def _dispatch_body(normed_hbm, d0_hbm, d1_hbm, out_hbm,
                   idx0_v, idx1_v, rows_v, lsem, sem):
    wid = lax.axis_index("s") * NC + lax.axis_index("c")
    lds = [pltpu.async_copy(
        normed_hbm.at[pl.ds(wid * TPW + c * SC_, SC_)],
        rows_v.at[pl.ds(c * SC_, SC_)], lsem) for c in range(SNCH)]
    pltpu.sync_copy(d0_hbm.at[wid], idx0_v)
    pltpu.sync_copy(d1_hbm.at[wid], idx1_v)
    cps = []
    for c in range(SNCH):
        lds[c].wait()
        src = rows_v.at[pl.ds(c * SC_, SC_)]
        cps.append(pltpu.async_copy(src, out_hbm.at[idx0_v.at[c]], sem))
        cps.append(pltpu.async_copy(src, out_hbm.at[idx1_v.at[c]], sem))
    for cp in cps:
        cp.wait()


@functools.lru_cache(maxsize=None)
def _make_k2():
    return functools.partial(
        pl.kernel,
        out_type=jax.ShapeDtypeStruct((NP, D), jnp.float32),
        mesh=plsc.VectorSubcoreMesh(core_axis_name="c", subcore_axis_name="s",
                                    num_cores=NC, num_subcores=NS),
        scratch_types=[
            pltpu.VMEM((SNCH, SC_), jnp.int32),
            pltpu.VMEM((SNCH, SC_), jnp.int32),
            pltpu.VMEM((TPW, D), jnp.float32),
            pltpu.SemaphoreType.DMA,
            pltpu.SemaphoreType.DMA,
        ],
    )(_dispatch_body)


def _k3_body(be_ref, x_ref, nw_ref, w1_ref, w2_ref, wo_ref, o_ref):
    raw = be_ref[pl.program_id(0)]
    e = lax.rem(raw, E)

    @pl.when(raw < E)
    def _():
        xr = x_ref[...]
        var = jnp.mean(xr * xr, axis=-1, keepdims=True)
        x = nw_ref[...] * (xr * lax.rsqrt(var + EPS))
        dn = (((1,), (1,)), ((), ()))
        h1 = lax.dot_general(x, w1_ref[e], dn)
        h2 = lax.dot_general(x, w2_ref[e], dn)
        inter = h1 * (h2 * jax.nn.sigmoid(h2))
        o_ref[...] = lax.dot_general(inter, wo_ref[e], dn)


_k3 = pl.pallas_call(
    _k3_body,
    grid_spec=pltpu.PrefetchScalarGridSpec(
        num_scalar_prefetch=1,
        grid=(NB,),
        in_specs=[
            pl.BlockSpec((BM, D), lambda b, be: (jnp.where(be[b] >= E, 0, b), 0)),
            pl.BlockSpec((1, D), lambda b, be: (0, 0)),
            pl.BlockSpec((E, F, D), lambda b, be: (0, 0, 0)),
            pl.BlockSpec((E, F, D), lambda b, be: (0, 0, 0)),
            pl.BlockSpec((E, D, F), lambda b, be: (0, 0, 0)),
        ],
        out_specs=pl.BlockSpec((BM, D), lambda b, be: (b, 0)),
    ),
    out_shape=jax.ShapeDtypeStruct((NP, D), jnp.float32),
)


def _pairgather_body(rows_hbm, dga_hbm, out_hbm, idx_v, rows_v, sem):
    wid = lax.axis_index("s") * NC + lax.axis_index("c")
    pltpu.sync_copy(dga_hbm.at[wid], idx_v)
    base = wid * PPW
    for c in range(GNCH):
        pltpu.async_copy(rows_hbm.at[idx_v.at[c]], rows_v, sem).wait()
        pltpu.sync_copy(rows_v, out_hbm.at[pl.ds(base + c * GC, GC)])


@functools.lru_cache(maxsize=None)
def _make_k4():
    return functools.partial(
        pl.kernel,
        out_type=jax.ShapeDtypeStruct((T * TK, D), jnp.float32),
        mesh=plsc.VectorSubcoreMesh(core_axis_name="c", subcore_axis_name="s",
                                    num_cores=NC, num_subcores=NS),
        scratch_types=[
            pltpu.VMEM((GNCH, GC), jnp.int32),
            pltpu.VMEM((GC, D), jnp.float32),
            pltpu.SemaphoreType.DMA,
        ],
    )(_pairgather_body)


def _k5_body(pairs_ref, base_ref, p1_ref, p2_ref, o_ref):
    ident = (lax.broadcasted_iota(jnp.int32, (BT, BT), 0)
             == lax.broadcasted_iota(jnp.int32, (BT, BT), 1)
             ).astype(jnp.float32)
    dnl = (((1,), (1,)), ((), ()))
    w0 = lax.dot_general(ident, p1_ref[...].reshape(1, BT), dnl,
                         precision=lax.Precision.HIGHEST)
    w1 = lax.dot_general(ident, p2_ref[...].reshape(1, BT), dnl,
                         precision=lax.Precision.HIGHEST)
    xr = pairs_ref[...].reshape(BT, TK, D)
    o_ref[...] = base_ref[...] + w0 * xr[:, 0, :] + w1 * xr[:, 1, :]


_k5 = pl.pallas_call(
    _k5_body,
    grid=(_NBT,),
    in_specs=[
        pl.BlockSpec((TK * BT, D), lambda b: (b, 0)),
        pl.BlockSpec((BT, D), lambda b: (b, 0)),
        pl.BlockSpec((1, 1, BT), lambda b: (b, 0, 0)),
        pl.BlockSpec((1, 1, BT), lambda b: (b, 0, 0)),
    ],
    out_specs=pl.BlockSpec((BT, D), lambda b: (b, 0)),
    out_shape=jax.ShapeDtypeStruct((T, D), jnp.float32),
)


@jax.jit
def kernel(hidden_states, norm_w, gate_w, w1, w2, wo, sw1, sw2, swo):
    flat = hidden_states.reshape(T, D)
    nw2 = norm_w.reshape(1, D)
    i1o, i2o, p1o, p2o = _k1a(flat, nw2, gate_w)
    base = _k1b(flat, sw1, sw2, swo)

    d0, d1, dga, beo = _k15(i1o.reshape(64, 32), i2o.reshape(64, 32))
    block_e = beo[:NB, 0]

    dispatched = _make_k2()(flat, d0, d1)
    out_rows = _k3(block_e, dispatched, nw2, w1, w2, wo)
    pairs = _make_k4()(out_rows, dga)
    y = _k5(pairs, base, p1o, p2o)
    return y.reshape(hidden_states.shape)
